# baseline (speedup 1.0000x reference)
import functools

import jax
import jax.numpy as jnp
from jax import lax
from jax.experimental import pallas as pl
from jax.experimental.pallas import tpu as pltpu
from jax.experimental.pallas import tpu_sc as plsc

_K = 16
_ROWS = 256
_BM = 1024
_BIG = 3.0e38


def _fused_body(n, dim, p_ref, pn_ref, q_ref, wh_ref, wl_ref, b_ref,
                idx_ref, a_ref, c_ref):
    r = pl.program_id(0)
    x = q_ref[...]
    xh = x.astype(jnp.bfloat16)
    xl = (x - xh.astype(jnp.float32)).astype(jnp.bfloat16)
    wh = wh_ref[...]
    wl = wl_ref[...]
    acc = (jnp.dot(xl, wl, preferred_element_type=jnp.float32)
           + jnp.dot(xh, wl, preferred_element_type=jnp.float32)
           + jnp.dot(xl, wh, preferred_element_type=jnp.float32)
           + jnp.dot(xh, wh, preferred_element_type=jnp.float32))
    a_ref[...] = acc[:, :dim]
    c_ref[...] = acc[:, dim:] + b_ref[...]
    x = p_ref[0, :]
    y = p_ref[1, :]
    z = p_ref[2, :]
    ss = x * x + y * y + z * z
    r0 = r * _ROWS
    xi = p_ref[0, pl.ds(r0, _ROWS)]
    yi = p_ref[1, pl.ds(r0, _ROWS)]
    zi = p_ref[2, pl.ds(r0, _ROWS)]
    ssi = xi * xi + yi * yi + zi * zi
    cross = jnp.dot(pn_ref[...].astype(jnp.bfloat16),
                    p_ref[...].astype(jnp.bfloat16),
                    preferred_element_type=jnp.float32)
    dist = (-2.0 * cross + ssi[:, None]) + ss[None, :]
    iota = lax.broadcasted_iota(jnp.int32, (_ROWS, n), 1).astype(jnp.float32)
    for kk in range(_K):
        m = jnp.min(dist, axis=1)
        eq = dist == m[:, None]
        cand = jnp.where(eq, iota, float(n))
        j = jnp.min(cand, axis=1)
        idx_ref[:, kk] = j.astype(jnp.int32)
        if kk + 1 < _K:
            dist = jnp.where(eq, _BIG, dist)


def _fused_tc(p2, pn8, qb, wh, wl, bias2):
    _, n = p2.shape
    dim = qb.shape[1]
    grid = (n // _ROWS,)
    return pl.pallas_call(
        functools.partial(_fused_body, n, dim),
        grid=grid,
        in_specs=[
            pl.BlockSpec((8, n), lambda r: (0, 0)),
            pl.BlockSpec((_ROWS, 8), lambda r: (r, 0)),
            pl.BlockSpec((_ROWS, dim), lambda r: (r, 0)),
            pl.BlockSpec((dim, 2 * dim), lambda r: (0, 0)),
            pl.BlockSpec((dim, 2 * dim), lambda r: (0, 0)),
            pl.BlockSpec((1, dim), lambda r: (0, 0)),
        ],
        out_specs=[
            pl.BlockSpec((_ROWS, _K), lambda r: (r, 0)),
            pl.BlockSpec((_ROWS, dim), lambda r: (r, 0)),
            pl.BlockSpec((_ROWS, dim), lambda r: (r, 0)),
        ],
        out_shape=[
            jax.ShapeDtypeStruct((n, _K), jnp.int32),
            jax.ShapeDtypeStruct((n, dim), jnp.float32),
            jax.ShapeDtypeStruct((n, dim), jnp.float32),
        ],
    )(p2, pn8, qb, wh, wl, bias2)


def _gather_max_sc(a2, idxg, cc):
    m, dim = cc.shape
    nw = 32
    rows_per_w = m // nw
    out_chunk = 8
    g = out_chunk * _K
    n_chunks = rows_per_w // out_chunk
    mesh = plsc.VectorSubcoreMesh(core_axis_name="c", subcore_axis_name="s")

    @functools.partial(
        pl.kernel,
        out_type=jax.ShapeDtypeStruct((m, dim), jnp.float32),
        mesh=mesh,
        scratch_types=[
            pltpu.VMEM((rows_per_w * _K,), jnp.int32),
            pltpu.VMEM((g, dim), jnp.float32),
            pltpu.VMEM((g, dim), jnp.float32),
            pltpu.VMEM((rows_per_w, dim), jnp.float32),
            pltpu.VMEM((rows_per_w, dim), jnp.float32),
            pltpu.SemaphoreType.DMA,
            pltpu.SemaphoreType.DMA,
            pltpu.SemaphoreType.DMA,
        ],
    )
    def sc_kernel(a_hbm, idx_hbm, cc_hbm, out_hbm, idx_v, rows_v0, rows_v1,
                  cc_v, out_v, sem_g0, sem_g1, sem_cc):
        wid = lax.axis_index("s") * 2 + lax.axis_index("c")
        base = wid * rows_per_w
        rows_b = (rows_v0, rows_v1)
        sems_b = (sem_g0, sem_g1)

        def gather_start(ci, par):
            pltpu.async_copy(a_hbm.at[idx_v.at[pl.ds(ci * g, g)]],
                             rows_b[par], sems_b[par])

        def gather_wait(ci, par):
            pltpu.make_async_copy(a_hbm.at[idx_v.at[pl.ds(ci * g, g)]],
                                  rows_b[par], sems_b[par]).wait()

        def compute(ci, par):
            rv = rows_b[par]

            @pl.loop(0, out_chunk)
            def _row(i):
                for c0 in range(0, dim, 16):
                    sl = pl.ds(c0, 16)
                    vals = [rv[pl.ds(i * _K + rr, 1), sl] for rr in range(_K)]
                    while len(vals) > 1:
                        vals = [jnp.maximum(vals[a], vals[a + 1])
                                for a in range(0, len(vals), 2)]
                    acc = vals[0] + cc_v[pl.ds(ci * out_chunk + i, 1), sl]
                    out_v[pl.ds(ci * out_chunk + i, 1), sl] = jnp.where(
                        acc >= 0.0, acc, 0.2 * acc)

        pltpu.sync_copy(idx_hbm.at[pl.ds(base * _K, rows_per_w * _K)], idx_v)
        gather_start(0, 0)
        pltpu.async_copy(cc_hbm.at[pl.ds(base, rows_per_w)], cc_v, sem_cc)
        pltpu.make_async_copy(cc_hbm.at[pl.ds(base, rows_per_w)], cc_v,
                              sem_cc).wait()

        @pl.loop(0, n_chunks // 2)
        def _pair(h):
            ci0 = h * 2
            gather_start(ci0 + 1, 1)
            gather_wait(ci0, 0)
            compute(ci0, 0)

            @pl.when(ci0 + 2 < n_chunks)
            def _():
                gather_start(ci0 + 2, 0)

            gather_wait(ci0 + 1, 1)
            compute(ci0 + 1, 1)

        pltpu.sync_copy(out_v, out_hbm.at[pl.ds(base, rows_per_w)])

    return sc_kernel(a2, idxg, cc)


def kernel(q, q_pos, W, b):
    bsz, n, dim = q.shape

    w1 = W[:, :dim]
    w2 = W[:, dim:]
    wcat = jnp.concatenate([w1.T, (w2 - w1).T], axis=1)
    wh = wcat.astype(jnp.bfloat16)
    wl = (wcat - wh.astype(jnp.float32)).astype(jnp.bfloat16)
    bias2 = b.reshape(1, dim)

    p = jnp.transpose(q_pos, (0, 2, 1))
    p8 = jnp.concatenate([p, jnp.zeros((bsz, 5, n), jnp.float32)], axis=1)
    pn8 = jnp.concatenate([q_pos, jnp.zeros((bsz, n, 5), jnp.float32)],
                          axis=2)

    outs = []
    for bb in range(bsz):
        idx_b, a_b, c_b = _fused_tc(p8[bb], pn8[bb], q[bb], wh, wl, bias2)
        outs.append(c_b if False else _gather_max_sc(a_b, idx_b.reshape(n * _K), c_b))
    return jnp.stack(outs)


# speedup vs baseline: 1.0063x; 1.0063x over previous
import functools

import jax
import jax.numpy as jnp
from jax import lax
from jax.experimental import pallas as pl
from jax.experimental.pallas import tpu as pltpu
from jax.experimental.pallas import tpu_sc as plsc

_K = 16
_ROWS = 256
_BM = 1024
_BIG = 3.0e38


def _fused_body(n, dim, p_ref, pn_ref, q_ref, wh_ref, wl_ref, b_ref,
                idx_ref, a_ref, c_ref):
    r = pl.program_id(0)
    x = q_ref[...]
    xh = x.astype(jnp.bfloat16)
    xl = (x - xh.astype(jnp.float32)).astype(jnp.bfloat16)
    wh = wh_ref[...]
    wl = wl_ref[...]
    acc = (jnp.dot(xh, wl, preferred_element_type=jnp.float32)
           + jnp.dot(xl, wh, preferred_element_type=jnp.float32)
           + jnp.dot(xh, wh, preferred_element_type=jnp.float32))
    a_ref[...] = acc[:, :dim]
    c_ref[...] = acc[:, dim:] + b_ref[...]
    x = p_ref[0, :]
    y = p_ref[1, :]
    z = p_ref[2, :]
    ss = x * x + y * y + z * z
    r0 = r * _ROWS
    xi = p_ref[0, pl.ds(r0, _ROWS)]
    yi = p_ref[1, pl.ds(r0, _ROWS)]
    zi = p_ref[2, pl.ds(r0, _ROWS)]
    ssi = xi * xi + yi * yi + zi * zi
    cross = jnp.dot(pn_ref[...].astype(jnp.bfloat16),
                    p_ref[...].astype(jnp.bfloat16),
                    preferred_element_type=jnp.float32)
    dist = (-2.0 * cross + ssi[:, None]) + ss[None, :]
    iota = lax.broadcasted_iota(jnp.int32, (_ROWS, n), 1).astype(jnp.float32)
    for kk in range(_K):
        m = jnp.min(dist, axis=1)
        eq = dist == m[:, None]
        cand = jnp.where(eq, iota, float(n))
        j = jnp.min(cand, axis=1)
        idx_ref[:, kk] = j.astype(jnp.int32)
        if kk + 1 < _K:
            dist = jnp.where(eq, _BIG, dist)


def _fused_tc(p2, pn8, qb, wh, wl, bias2):
    _, n = p2.shape
    dim = qb.shape[1]
    grid = (n // _ROWS,)
    return pl.pallas_call(
        functools.partial(_fused_body, n, dim),
        grid=grid,
        in_specs=[
            pl.BlockSpec((8, n), lambda r: (0, 0)),
            pl.BlockSpec((_ROWS, 8), lambda r: (r, 0)),
            pl.BlockSpec((_ROWS, dim), lambda r: (r, 0)),
            pl.BlockSpec((dim, 2 * dim), lambda r: (0, 0)),
            pl.BlockSpec((dim, 2 * dim), lambda r: (0, 0)),
            pl.BlockSpec((1, dim), lambda r: (0, 0)),
        ],
        out_specs=[
            pl.BlockSpec((_ROWS, _K), lambda r: (r, 0)),
            pl.BlockSpec((_ROWS, dim), lambda r: (r, 0)),
            pl.BlockSpec((_ROWS, dim), lambda r: (r, 0)),
        ],
        out_shape=[
            jax.ShapeDtypeStruct((n, _K), jnp.int32),
            jax.ShapeDtypeStruct((n, dim), jnp.float32),
            jax.ShapeDtypeStruct((n, dim), jnp.float32),
        ],
    )(p2, pn8, qb, wh, wl, bias2)


def _gather_max_sc(a2, idxg, cc):
    m, dim = cc.shape
    nw = 32
    rows_per_w = m // nw
    out_chunk = 8
    g = out_chunk * _K
    n_chunks = rows_per_w // out_chunk
    mesh = plsc.VectorSubcoreMesh(core_axis_name="c", subcore_axis_name="s")

    @functools.partial(
        pl.kernel,
        out_type=jax.ShapeDtypeStruct((m, dim), jnp.float32),
        mesh=mesh,
        scratch_types=[
            pltpu.VMEM((rows_per_w * _K,), jnp.int32),
            pltpu.VMEM((g, dim), jnp.float32),
            pltpu.VMEM((g, dim), jnp.float32),
            pltpu.VMEM((rows_per_w, dim), jnp.float32),
            pltpu.VMEM((rows_per_w, dim), jnp.float32),
            pltpu.SemaphoreType.DMA,
            pltpu.SemaphoreType.DMA,
            pltpu.SemaphoreType.DMA,
        ],
    )
    def sc_kernel(a_hbm, idx_hbm, cc_hbm, out_hbm, idx_v, rows_v0, rows_v1,
                  cc_v, out_v, sem_g0, sem_g1, sem_cc):
        wid = lax.axis_index("s") * 2 + lax.axis_index("c")
        base = wid * rows_per_w
        rows_b = (rows_v0, rows_v1)
        sems_b = (sem_g0, sem_g1)

        def gather_start(ci, par):
            pltpu.async_copy(a_hbm.at[idx_v.at[pl.ds(ci * g, g)]],
                             rows_b[par], sems_b[par])

        def gather_wait(ci, par):
            pltpu.make_async_copy(a_hbm.at[idx_v.at[pl.ds(ci * g, g)]],
                                  rows_b[par], sems_b[par]).wait()

        def compute(ci, par):
            rv = rows_b[par]

            @pl.loop(0, out_chunk)
            def _row(i):
                for c0 in range(0, dim, 16):
                    sl = pl.ds(c0, 16)
                    vals = [rv[pl.ds(i * _K + rr, 1), sl] for rr in range(_K)]
                    while len(vals) > 1:
                        vals = [jnp.maximum(vals[a], vals[a + 1])
                                for a in range(0, len(vals), 2)]
                    acc = vals[0] + cc_v[pl.ds(ci * out_chunk + i, 1), sl]
                    out_v[pl.ds(ci * out_chunk + i, 1), sl] = jnp.where(
                        acc >= 0.0, acc, 0.2 * acc)

        pltpu.sync_copy(idx_hbm.at[pl.ds(base * _K, rows_per_w * _K)], idx_v)
        gather_start(0, 0)
        pltpu.async_copy(cc_hbm.at[pl.ds(base, rows_per_w)], cc_v, sem_cc)
        pltpu.make_async_copy(cc_hbm.at[pl.ds(base, rows_per_w)], cc_v,
                              sem_cc).wait()

        @pl.loop(0, n_chunks // 2)
        def _pair(h):
            ci0 = h * 2
            gather_start(ci0 + 1, 1)
            gather_wait(ci0, 0)
            compute(ci0, 0)

            @pl.when(ci0 + 2 < n_chunks)
            def _():
                gather_start(ci0 + 2, 0)

            gather_wait(ci0 + 1, 1)
            compute(ci0 + 1, 1)

        pltpu.sync_copy(out_v, out_hbm.at[pl.ds(base, rows_per_w)])

    return sc_kernel(a2, idxg, cc)


def kernel(q, q_pos, W, b):
    bsz, n, dim = q.shape

    w1 = W[:, :dim]
    w2 = W[:, dim:]
    wcat = jnp.concatenate([w1.T, (w2 - w1).T], axis=1)
    wh = wcat.astype(jnp.bfloat16)
    wl = (wcat - wh.astype(jnp.float32)).astype(jnp.bfloat16)
    bias2 = b.reshape(1, dim)

    p = jnp.transpose(q_pos, (0, 2, 1))
    p8 = jnp.concatenate([p, jnp.zeros((bsz, 5, n), jnp.float32)], axis=1)
    pn8 = jnp.concatenate([q_pos, jnp.zeros((bsz, n, 5), jnp.float32)],
                          axis=2)

    outs = []
    for bb in range(bsz):
        idx_b, a_b, c_b = _fused_tc(p8[bb], pn8[bb], q[bb], wh, wl, bias2)
        outs.append(c_b if False else _gather_max_sc(a_b, idx_b.reshape(n * _K), c_b))
    return jnp.stack(outs)
